# baseline (speedup 1.0000x reference)
import jax
import jax.numpy as jnp
from jax.experimental import pallas as pl
from jax.experimental.pallas import tpu as pltpu

E = 8
H = 1024
I_MOE = 512
I_SHARED = 1024
SH_CHUNK = I_SHARED // E
N_STEPS = E + 2


def _dot_t(a, b):
    return jax.lax.dot_general(
        a, b, (((1,), (1,)), ((), ())), preferred_element_type=jnp.float32
    )


def _dot_t_bf16(a, b):
    return jax.lax.dot_general(
        a.astype(jnp.bfloat16), b.astype(jnp.bfloat16),
        (((1,), (1,)), ((), ())), preferred_element_type=jnp.float32,
    )


def _silu(x):
    return x * jax.nn.sigmoid(x)


def _moe_kernel(x_ref, gate_w_ref, gp_ref, up_ref, dp_ref,
                sg_ref, su_ref, sd_ref, seg_ref,
                out_ref, combine_ref, gv_ref, gbuf, ubuf, abuf):
    s = pl.program_id(0)
    x = x_ref[...]
    t = x.shape[0]

    @pl.when(s == 0)
    def _router():
        logits = _dot_t(x, gate_w_ref[...])
        m = jnp.max(logits, axis=-1, keepdims=True)
        p = jnp.exp(logits - m)
        p = p / jnp.sum(p, axis=-1, keepdims=True)

        e_iota = jax.lax.broadcasted_iota(jnp.int32, (t, E), 1)
        w1 = jnp.max(p, axis=-1, keepdims=True)
        i1 = jnp.min(jnp.where(p == w1, e_iota, E), axis=-1, keepdims=True)
        m1 = e_iota == i1
        p2 = jnp.where(m1, -1.0, p)
        w2 = jnp.max(p2, axis=-1, keepdims=True)
        i2 = jnp.min(jnp.where(p2 == w2, e_iota, E), axis=-1, keepdims=True)
        m2 = e_iota == i2
        combine = jnp.where(m1, w1, 0.0) + jnp.where(m2, w2, 0.0)
        combine_ref[...] = combine / (w1 + w2)

        gv_ref[...] = jax.nn.sigmoid(_dot_t(x, seg_ref[...]))

    @pl.when(s < E)
    def _stage1_gu():
        gbuf[s % 2] = _dot_t_bf16(x, gp_ref[0]).astype(jnp.bfloat16)
        ubuf[s % 2] = _dot_t_bf16(x, up_ref[0]).astype(jnp.bfloat16)

    @pl.when((s >= 1) & (s <= E))
    def _stage2_act():
        e2 = s - 1
        g = gbuf[e2 % 2][...].astype(jnp.float32)
        u = ubuf[e2 % 2][...].astype(jnp.float32)
        act = _silu(g) * u
        combine = combine_ref[...]
        col = jax.lax.broadcasted_iota(jnp.int32, (t, E), 1) == e2
        w_e = jnp.sum(jnp.where(col, combine, 0.0), axis=-1, keepdims=True)
        abuf[e2 % 2] = (act * w_e).astype(jnp.bfloat16)

    @pl.when(s >= 2)
    def _stage3_down_and_shared():
        e3 = s - 2
        a = abuf[e3 % 2][...]
        contrib = _dot_t_bf16(a, dp_ref[0])

        sg = _dot_t_bf16(x, sg_ref[...])
        su = _dot_t_bf16(x, su_ref[...])
        sh = _dot_t_bf16(_silu(sg) * su, sd_ref[...])
        contrib = contrib + gv_ref[...] * sh

        @pl.when(s == 2)
        def _init():
            out_ref[...] = contrib

        @pl.when(s > 2)
        def _acc():
            out_ref[...] += contrib


def kernel(hidden_states, gate_w, gate_proj_w, up_proj_w, down_proj_w,
           shared_gate_w, shared_up_w, shared_down_w, shared_expert_gate_w):
    b, s, h = hidden_states.shape
    x = hidden_states.reshape(-1, h)
    t = x.shape[0]

    def _e1(i):
        return jnp.minimum(i, E - 1)

    def _e3(i):
        return jnp.clip(i - 2, 0, E - 1)

    out = pl.pallas_call(
        _moe_kernel,
        grid=(N_STEPS,),
        in_specs=[
            pl.BlockSpec((t, h), lambda i: (0, 0)),
            pl.BlockSpec((E, h), lambda i: (0, 0)),
            pl.BlockSpec((1, I_MOE, h), lambda i: (_e1(i), 0, 0)),
            pl.BlockSpec((1, I_MOE, h), lambda i: (_e1(i), 0, 0)),
            pl.BlockSpec((1, h, I_MOE), lambda i: (_e3(i), 0, 0)),
            pl.BlockSpec((SH_CHUNK, h), lambda i: (_e3(i), 0)),
            pl.BlockSpec((SH_CHUNK, h), lambda i: (_e3(i), 0)),
            pl.BlockSpec((h, SH_CHUNK), lambda i: (0, _e3(i))),
            pl.BlockSpec((1, h), lambda i: (0, 0)),
        ],
        out_specs=pl.BlockSpec((t, h), lambda i: (0, 0)),
        out_shape=jax.ShapeDtypeStruct((t, h), jnp.float32),
        scratch_shapes=[
            pltpu.VMEM((t, E), jnp.float32),
            pltpu.VMEM((t, 1), jnp.float32),
            pltpu.VMEM((2, t, I_MOE), jnp.bfloat16),
            pltpu.VMEM((2, t, I_MOE), jnp.bfloat16),
            pltpu.VMEM((2, t, I_MOE), jnp.bfloat16),
        ],
    )(x, gate_w, gate_proj_w, up_proj_w, down_proj_w,
      shared_gate_w, shared_up_w, shared_down_w, shared_expert_gate_w)

    return out.reshape(b, s, h)


# speedup vs baseline: 1.0551x; 1.0551x over previous
import jax
import jax.numpy as jnp
from jax.experimental import pallas as pl
from jax.experimental.pallas import tpu as pltpu

E = 8
H = 1024
I_MOE = 512
I_SHARED = 1024
SH_CHUNK = 2 * (I_SHARED // E)
N_BODIES = (E + 2) // 2


def _dot_t(a, b):
    return jax.lax.dot_general(
        a, b, (((1,), (1,)), ((), ())), preferred_element_type=jnp.float32
    )


def _bf(v):
    return v.astype(jnp.bfloat16)


def _silu(x):
    return x * jax.nn.sigmoid(x)


def _moe_kernel(x_ref, gate_w_ref, gp_ref, up_ref, dp_ref,
                sg_ref, su_ref, sd_ref, seg_ref,
                out_ref, combine_ref, gv_ref, xb_ref,
                gbuf0, gbuf1, ubuf0, ubuf1, abuf0, abuf1):
    k = pl.program_id(0)
    t = combine_ref.shape[0]
    e_iota = jax.lax.broadcasted_iota(jnp.int32, (t, E), 1)

    @pl.when(k == 0)
    def _router():
        x = x_ref[...]
        xb_ref[...] = _bf(x)

        logits = _dot_t(x, gate_w_ref[...])
        m = jnp.max(logits, axis=-1, keepdims=True)
        p = jnp.exp(logits - m)
        p = p / jnp.sum(p, axis=-1, keepdims=True)

        w1 = jnp.max(p, axis=-1, keepdims=True)
        i1 = jnp.min(jnp.where(p == w1, e_iota, E), axis=-1, keepdims=True)
        m1 = e_iota == i1
        p2 = jnp.where(m1, -1.0, p)
        w2 = jnp.max(p2, axis=-1, keepdims=True)
        i2 = jnp.min(jnp.where(p2 == w2, e_iota, E), axis=-1, keepdims=True)
        m2 = e_iota == i2
        combine = jnp.where(m1, w1, 0.0) + jnp.where(m2, w2, 0.0)
        combine_ref[...] = combine / (w1 + w2)

        gv_ref[...] = jax.nn.sigmoid(_dot_t(x, seg_ref[...]))

    xb = xb_ref[...]
    combine = combine_ref[...]

    def _act(g_bufref, u_bufref, e2):
        g = g_bufref[...].astype(jnp.float32)
        u = u_bufref[...].astype(jnp.float32)
        act = _silu(g) * u
        w_e = jnp.sum(jnp.where(e_iota == e2, combine, 0.0),
                      axis=-1, keepdims=True)
        return _bf(act * w_e)

    gbuf0[...] = _bf(_dot_t(xb, _bf(gp_ref[0])))
    ubuf0[...] = _bf(_dot_t(xb, _bf(up_ref[0])))
    abuf1[...] = _act(gbuf1, ubuf1, 2 * k - 1)
    contrib = _dot_t(abuf0[...], _bf(dp_ref[0]))

    gbuf1[...] = _bf(_dot_t(xb, _bf(gp_ref[1])))
    ubuf1[...] = _bf(_dot_t(xb, _bf(up_ref[1])))
    abuf0[...] = _act(gbuf0, ubuf0, 2 * k)
    contrib = contrib + _dot_t(abuf1[...], _bf(dp_ref[1]))

    sg = _dot_t(xb, _bf(sg_ref[...]))
    su = _dot_t(xb, _bf(su_ref[...]))
    sh = _dot_t(_bf(_silu(sg) * su), _bf(sd_ref[...]))
    contrib = contrib + gv_ref[...] * sh

    @pl.when(k == 1)
    def _init():
        out_ref[...] = contrib

    @pl.when(k > 1)
    def _acc():
        out_ref[...] += contrib


def kernel(hidden_states, gate_w, gate_proj_w, up_proj_w, down_proj_w,
           shared_gate_w, shared_up_w, shared_down_w, shared_expert_gate_w):
    b, s, h = hidden_states.shape
    x = hidden_states.reshape(-1, h)
    t = x.shape[0]

    ne = E // 2

    def _i1(i):
        return jnp.minimum(i, ne - 1)

    def _i3(i):
        return jnp.clip(i - 1, 0, ne - 1)

    out = pl.pallas_call(
        _moe_kernel,
        grid=(N_BODIES,),
        in_specs=[
            pl.BlockSpec((t, h), lambda i: (0, 0)),
            pl.BlockSpec((E, h), lambda i: (0, 0)),
            pl.BlockSpec((2, I_MOE, h), lambda i: (_i1(i), 0, 0)),
            pl.BlockSpec((2, I_MOE, h), lambda i: (_i1(i), 0, 0)),
            pl.BlockSpec((2, h, I_MOE), lambda i: (_i3(i), 0, 0)),
            pl.BlockSpec((SH_CHUNK, h), lambda i: (_i3(i), 0)),
            pl.BlockSpec((SH_CHUNK, h), lambda i: (_i3(i), 0)),
            pl.BlockSpec((h, SH_CHUNK), lambda i: (0, _i3(i))),
            pl.BlockSpec((1, h), lambda i: (0, 0)),
        ],
        out_specs=pl.BlockSpec((t, h), lambda i: (0, 0)),
        out_shape=jax.ShapeDtypeStruct((t, h), jnp.float32),
        scratch_shapes=[
            pltpu.VMEM((t, E), jnp.float32),
            pltpu.VMEM((t, 1), jnp.float32),
            pltpu.VMEM((t, H), jnp.bfloat16),
            pltpu.VMEM((t, I_MOE), jnp.bfloat16),
            pltpu.VMEM((t, I_MOE), jnp.bfloat16),
            pltpu.VMEM((t, I_MOE), jnp.bfloat16),
            pltpu.VMEM((t, I_MOE), jnp.bfloat16),
            pltpu.VMEM((t, I_MOE), jnp.bfloat16),
            pltpu.VMEM((t, I_MOE), jnp.bfloat16),
        ],
    )(x, gate_w, gate_proj_w, up_proj_w, down_proj_w,
      shared_gate_w, shared_up_w, shared_down_w, shared_expert_gate_w)

    return out.reshape(b, s, h)


# speedup vs baseline: 1.0825x; 1.0259x over previous
import jax
import jax.numpy as jnp
from jax.experimental import pallas as pl
from jax.experimental.pallas import tpu as pltpu

E = 8
H = 1024
I_MOE = 512
I_SHARED = 1024
SH_CHUNK = I_SHARED // E


def _dot_t(a, b):
    return jax.lax.dot_general(
        a, b, (((1,), (1,)), ((), ())), preferred_element_type=jnp.float32
    )


def _bf(v):
    return v.astype(jnp.bfloat16)


def _silu(x):
    return x * jax.nn.sigmoid(x)


def _moe_kernel(x_ref, gate_w_ref, gp_ref, up_ref, dp_ref,
                sg_ref, su_ref, sd_ref, seg_ref,
                out_ref, combine_ref, gv_ref):
    e = pl.program_id(0)
    x = x_ref[...]
    t = x.shape[0]
    e_iota = jax.lax.broadcasted_iota(jnp.int32, (t, E), 1)

    @pl.when(e == 0)
    def _router():
        logits = _dot_t(x, gate_w_ref[...])
        m = jnp.max(logits, axis=-1, keepdims=True)
        p = jnp.exp(logits - m)
        p = p / jnp.sum(p, axis=-1, keepdims=True)

        w1 = jnp.max(p, axis=-1, keepdims=True)
        i1 = jnp.min(jnp.where(p == w1, e_iota, E), axis=-1, keepdims=True)
        m1 = e_iota == i1
        p2 = jnp.where(m1, -1.0, p)
        w2 = jnp.max(p2, axis=-1, keepdims=True)
        i2 = jnp.min(jnp.where(p2 == w2, e_iota, E), axis=-1, keepdims=True)
        m2 = e_iota == i2
        combine = jnp.where(m1, w1, 0.0) + jnp.where(m2, w2, 0.0)
        combine_ref[...] = combine / (w1 + w2)

        gv_ref[...] = jax.nn.sigmoid(_dot_t(x, seg_ref[...]))

    g = _dot_t(x, gp_ref[0])
    u = _dot_t(x, up_ref[0])
    act = _silu(g) * u
    w_e = jnp.sum(jnp.where(e_iota == e, combine_ref[...], 0.0),
                  axis=-1, keepdims=True)
    contrib = _dot_t(_bf(act * w_e), _bf(dp_ref[0]))

    sg = _dot_t(x, sg_ref[...])
    su = _dot_t(x, su_ref[...])
    sh = _dot_t(_bf(_silu(sg) * su), _bf(sd_ref[...]))
    contrib = contrib + gv_ref[...] * sh

    @pl.when(e == 0)
    def _init():
        out_ref[...] = contrib

    @pl.when(e > 0)
    def _acc():
        out_ref[...] += contrib


def kernel(hidden_states, gate_w, gate_proj_w, up_proj_w, down_proj_w,
           shared_gate_w, shared_up_w, shared_down_w, shared_expert_gate_w):
    b, s, h = hidden_states.shape
    x = hidden_states.reshape(-1, h)
    t = x.shape[0]

    out = pl.pallas_call(
        _moe_kernel,
        grid=(E,),
        in_specs=[
            pl.BlockSpec((t, h), lambda i: (0, 0)),
            pl.BlockSpec((E, h), lambda i: (0, 0)),
            pl.BlockSpec((1, I_MOE, h), lambda i: (i, 0, 0)),
            pl.BlockSpec((1, I_MOE, h), lambda i: (i, 0, 0)),
            pl.BlockSpec((1, h, I_MOE), lambda i: (i, 0, 0)),
            pl.BlockSpec((SH_CHUNK, h), lambda i: (i, 0)),
            pl.BlockSpec((SH_CHUNK, h), lambda i: (i, 0)),
            pl.BlockSpec((h, SH_CHUNK), lambda i: (0, i)),
            pl.BlockSpec((1, h), lambda i: (0, 0)),
        ],
        out_specs=pl.BlockSpec((t, h), lambda i: (0, 0)),
        out_shape=jax.ShapeDtypeStruct((t, h), jnp.float32),
        scratch_shapes=[
            pltpu.VMEM((t, E), jnp.float32),
            pltpu.VMEM((t, 1), jnp.float32),
        ],
    )(x, gate_w, gate_proj_w, up_proj_w, down_proj_w,
      shared_gate_w, shared_up_w, shared_down_w, shared_expert_gate_w)

    return out.reshape(b, s, h)


# speedup vs baseline: 1.1979x; 1.1066x over previous
import jax
import jax.numpy as jnp
from jax.experimental import pallas as pl
from jax.experimental.pallas import tpu as pltpu

E = 8
H = 1024
I_MOE = 512
I_SHARED = 1024
SH_HALF = I_SHARED // 2
N_STEPS = E + 2


def _dot_t(a, b):
    return jax.lax.dot_general(
        a, b, (((1,), (1,)), ((), ())), preferred_element_type=jnp.float32
    )


def _bf(v):
    return v.astype(jnp.bfloat16)


def _silu(x):
    return x * jax.nn.sigmoid(x)


def _moe_kernel(x_ref, gate_w_ref, gp_ref, up_ref, dp_ref,
                sg_ref, su_ref, sd_ref, seg_ref,
                out_ref, combine_ref, gv_ref, xb_ref):
    s = pl.program_id(0)
    t = out_ref.shape[0]
    e_iota = jax.lax.broadcasted_iota(jnp.int32, (t, E), 1)

    @pl.when(s == 0)
    def _router():
        x = x_ref[...]
        xb_ref[...] = _bf(x)

        logits = _dot_t(x, gate_w_ref[...])
        m = jnp.max(logits, axis=-1, keepdims=True)
        p = jnp.exp(logits - m)
        p = p / jnp.sum(p, axis=-1, keepdims=True)

        w1 = jnp.max(p, axis=-1, keepdims=True)
        i1 = jnp.min(jnp.where(p == w1, e_iota, E), axis=-1, keepdims=True)
        m1 = e_iota == i1
        p2 = jnp.where(m1, -1.0, p)
        w2 = jnp.max(p2, axis=-1, keepdims=True)
        i2 = jnp.min(jnp.where(p2 == w2, e_iota, E), axis=-1, keepdims=True)
        m2 = e_iota == i2
        combine = jnp.where(m1, w1, 0.0) + jnp.where(m2, w2, 0.0)
        combine_ref[...] = combine / (w1 + w2)

        gv_ref[...] = jax.nn.sigmoid(_dot_t(x, seg_ref[...]))

    xb = xb_ref[...]

    @pl.when(s < E)
    def _expert():
        g = _dot_t(xb, _bf(gp_ref[0]))
        u = _dot_t(xb, _bf(up_ref[0]))
        act = _silu(g) * u
        w_e = jnp.sum(jnp.where(e_iota == s, combine_ref[...], 0.0),
                      axis=-1, keepdims=True)
        contrib = _dot_t(_bf(act * w_e), _bf(dp_ref[0]))

        @pl.when(s == 0)
        def _init():
            out_ref[...] = contrib

        @pl.when(s > 0)
        def _acc():
            out_ref[...] += contrib

    @pl.when(s >= E)
    def _shared_half():
        sg = _dot_t(xb, _bf(sg_ref[...]))
        su = _dot_t(xb, _bf(su_ref[...]))
        sh = _dot_t(_bf(_silu(sg) * su), _bf(sd_ref[...]))
        out_ref[...] += gv_ref[...] * sh


def kernel(hidden_states, gate_w, gate_proj_w, up_proj_w, down_proj_w,
           shared_gate_w, shared_up_w, shared_down_w, shared_expert_gate_w):
    b, s, h = hidden_states.shape
    x = hidden_states.reshape(-1, h)
    t = x.shape[0]

    def _ie(i):
        return jnp.minimum(i, E - 1)

    def _ish(i):
        return jnp.clip(i - E, 0, 1)

    out = pl.pallas_call(
        _moe_kernel,
        grid=(N_STEPS,),
        in_specs=[
            pl.BlockSpec((t, h), lambda i: (0, 0)),
            pl.BlockSpec((E, h), lambda i: (0, 0)),
            pl.BlockSpec((1, I_MOE, h), lambda i: (_ie(i), 0, 0)),
            pl.BlockSpec((1, I_MOE, h), lambda i: (_ie(i), 0, 0)),
            pl.BlockSpec((1, h, I_MOE), lambda i: (_ie(i), 0, 0)),
            pl.BlockSpec((SH_HALF, h), lambda i: (_ish(i), 0)),
            pl.BlockSpec((SH_HALF, h), lambda i: (_ish(i), 0)),
            pl.BlockSpec((h, SH_HALF), lambda i: (0, _ish(i))),
            pl.BlockSpec((1, h), lambda i: (0, 0)),
        ],
        out_specs=pl.BlockSpec((t, h), lambda i: (0, 0)),
        out_shape=jax.ShapeDtypeStruct((t, h), jnp.float32),
        scratch_shapes=[
            pltpu.VMEM((t, E), jnp.float32),
            pltpu.VMEM((t, 1), jnp.float32),
            pltpu.VMEM((t, H), jnp.bfloat16),
        ],
    )(x, gate_w, gate_proj_w, up_proj_w, down_proj_w,
      shared_gate_w, shared_up_w, shared_down_w, shared_expert_gate_w)

    return out.reshape(b, s, h)


# speedup vs baseline: 1.3100x; 1.0936x over previous
import jax
import jax.numpy as jnp
from jax.experimental import pallas as pl
from jax.experimental.pallas import tpu as pltpu

E = 8
H = 1024
I_MOE = 512
I_SHARED = 1024
SH_HALF = I_SHARED // 2
N_STEPS = E + 2


def _dot_t(a, b):
    return jax.lax.dot_general(
        a, b, (((1,), (1,)), ((), ())), preferred_element_type=jnp.float32
    )


def _bf(v):
    return v.astype(jnp.bfloat16)


def _silu(x):
    return x * jax.nn.sigmoid(x)


def _moe_kernel(x_ref, gate_w_ref, gp_ref, up_ref, dp_ref,
                sg_ref, su_ref, sd_ref, seg_ref,
                out_ref, combine_ref, xb_ref):
    s = pl.program_id(0)
    t = out_ref.shape[0]
    e_iota = jax.lax.broadcasted_iota(jnp.int32, (t, E), 1)

    @pl.when(s == 0)
    def _router():
        x = x_ref[...]
        xb_ref[...] = _bf(x)

        logits = _dot_t(x, gate_w_ref[...])
        m = jnp.max(logits, axis=-1, keepdims=True)
        p = jnp.exp(logits - m)
        p = p / jnp.sum(p, axis=-1, keepdims=True)

        w1 = jnp.max(p, axis=-1, keepdims=True)
        i1 = jnp.min(jnp.where(p == w1, e_iota, E), axis=-1, keepdims=True)
        m1 = e_iota == i1
        p2 = jnp.where(m1, -1.0, p)
        w2 = jnp.max(p2, axis=-1, keepdims=True)
        i2 = jnp.min(jnp.where(p2 == w2, e_iota, E), axis=-1, keepdims=True)
        m2 = e_iota == i2
        combine = jnp.where(m1, w1, 0.0) + jnp.where(m2, w2, 0.0)
        combine_ref[...] = combine / (w1 + w2)

        xbs = xb_ref[...]
        sg = _dot_t(xbs, _bf(sg_ref[...]))
        su = _dot_t(xbs, _bf(su_ref[...]))
        sh = _dot_t(_bf(_silu(sg) * su), _bf(sd_ref[...]))
        gv = jax.nn.sigmoid(_dot_t(x, seg_ref[...]))
        out_ref[...] = gv * sh

    xb = xb_ref[...]

    g = _dot_t(xb, _bf(gp_ref[0]))
    u = _dot_t(xb, _bf(up_ref[0]))
    act = _silu(g) * u
    w_e = jnp.sum(jnp.where(e_iota == s, combine_ref[...], 0.0),
                  axis=-1, keepdims=True)
    out_ref[...] += _dot_t(_bf(act * w_e), _bf(dp_ref[0]))


def kernel(hidden_states, gate_w, gate_proj_w, up_proj_w, down_proj_w,
           shared_gate_w, shared_up_w, shared_down_w, shared_expert_gate_w):
    b, s, h = hidden_states.shape
    x = hidden_states.reshape(-1, h)
    t = x.shape[0]

    out = pl.pallas_call(
        _moe_kernel,
        grid=(E,),
        in_specs=[
            pl.BlockSpec((t, h), lambda i: (0, 0)),
            pl.BlockSpec((E, h), lambda i: (0, 0)),
            pl.BlockSpec((1, I_MOE, h), lambda i: (i, 0, 0)),
            pl.BlockSpec((1, I_MOE, h), lambda i: (i, 0, 0)),
            pl.BlockSpec((1, h, I_MOE), lambda i: (i, 0, 0)),
            pl.BlockSpec((I_SHARED, h), lambda i: (0, 0)),
            pl.BlockSpec((I_SHARED, h), lambda i: (0, 0)),
            pl.BlockSpec((h, I_SHARED), lambda i: (0, 0)),
            pl.BlockSpec((1, h), lambda i: (0, 0)),
        ],
        out_specs=pl.BlockSpec((t, h), lambda i: (0, 0)),
        out_shape=jax.ShapeDtypeStruct((t, h), jnp.float32),
        scratch_shapes=[
            pltpu.VMEM((t, E), jnp.float32),
            pltpu.VMEM((t, H), jnp.bfloat16),
        ],
    )(x, gate_w, gate_proj_w, up_proj_w, down_proj_w,
      shared_gate_w, shared_up_w, shared_down_w, shared_expert_gate_w)

    return out.reshape(b, s, h)
